# baseline (device time: 101608 ns/iter reference)
import jax
import jax.numpy as jnp
from jax import lax
from jax.experimental import pallas as pl
from jax.experimental.pallas import tpu as pltpu

N_DEV = 8


def kernel(x, w_mat):
    m, k_shard = x.shape
    _, n = w_mat.shape
    m_chunk = m // N_DEV

    def body(
        x_ref, w_ref, out_ref,
        q_send, q_recv, s_send, s_recv,
        q_send_sems, q_recv_sems, s_send_sems, s_recv_sems,
    ):
        my = lax.axis_index("i")
        w = w_ref[:, :]

        rdmas = []
        for k in range(1, N_DEV):
            dst = (my + k) % N_DEV
            row0 = dst * m_chunk
            part = jnp.dot(
                x_ref[pl.ds(row0, m_chunk), :], w,
                preferred_element_type=jnp.float32,
            )
            absmax = jnp.maximum(
                jnp.max(jnp.abs(part), axis=1, keepdims=True), 1e-30
            )
            q_send[k - 1, :, :] = jnp.round(part * (127.0 / absmax)).astype(
                jnp.int8
            )
            s_send[k - 1, :, :] = absmax * (1.0 / 127.0)
            q_rdma = pltpu.make_async_remote_copy(
                src_ref=q_send.at[k - 1],
                dst_ref=q_recv.at[k - 1],
                send_sem=q_send_sems.at[k - 1],
                recv_sem=q_recv_sems.at[k - 1],
                device_id=(dst,),
                device_id_type=pl.DeviceIdType.MESH,
            )
            s_rdma = pltpu.make_async_remote_copy(
                src_ref=s_send.at[k - 1],
                dst_ref=s_recv.at[k - 1],
                send_sem=s_send_sems.at[k - 1],
                recv_sem=s_recv_sems.at[k - 1],
                device_id=(dst,),
                device_id_type=pl.DeviceIdType.MESH,
            )
            q_rdma.start()
            s_rdma.start()
            rdmas.append((q_rdma, s_rdma))

        out_ref[:, :] = jnp.dot(
            x_ref[pl.ds(my * m_chunk, m_chunk), :], w,
            preferred_element_type=jnp.float32,
        )

        for j in range(N_DEV - 1):
            q_rdma, s_rdma = rdmas[j]
            q_rdma.wait_recv()
            s_rdma.wait_recv()
            out_ref[:, :] = out_ref[:, :] + (
                q_recv[j, :, :].astype(jnp.float32) * s_recv[j, :, :]
            )

        out_ref[:, :] = jnp.maximum(out_ref[:, :], 0.0)

        for j in range(N_DEV - 1):
            rdmas[j][0].wait_send()
            rdmas[j][1].wait_send()

    return pl.pallas_call(
        body,
        out_shape=jax.ShapeDtypeStruct((m_chunk, n), jnp.float32),
        in_specs=[
            pl.BlockSpec(memory_space=pltpu.VMEM),
            pl.BlockSpec(memory_space=pltpu.VMEM),
        ],
        out_specs=pl.BlockSpec(memory_space=pltpu.VMEM),
        scratch_shapes=[
            pltpu.VMEM((N_DEV - 1, m_chunk, n), jnp.int8),
            pltpu.VMEM((N_DEV - 1, m_chunk, n), jnp.int8),
            pltpu.VMEM((N_DEV - 1, m_chunk, 1), jnp.float32),
            pltpu.VMEM((N_DEV - 1, m_chunk, 1), jnp.float32),
            pltpu.SemaphoreType.DMA((N_DEV - 1,)),
            pltpu.SemaphoreType.DMA((N_DEV - 1,)),
            pltpu.SemaphoreType.DMA((N_DEV - 1,)),
            pltpu.SemaphoreType.DMA((N_DEV - 1,)),
        ],
    )(x, w_mat)


# device time: 85666 ns/iter; 1.1861x vs baseline; 1.1861x over previous
import jax
import jax.numpy as jnp
from jax import lax
from jax.experimental import pallas as pl
from jax.experimental.pallas import tpu as pltpu

N_DEV = 8


def kernel(x, w_mat):
    m, k_shard = x.shape
    _, n = w_mat.shape
    m_chunk = m // N_DEV

    def body(
        x_ref, w_ref, out_ref,
        q_send, q_recv, s_send, s_recv,
        q_send_sems, q_recv_sems, s_send_sems, s_recv_sems,
    ):
        my = lax.axis_index("i")
        w = w_ref[:, :]

        rdmas = []
        for k in range(1, N_DEV):
            dst = (my + k) % N_DEV
            row0 = dst * m_chunk
            part = jnp.dot(
                x_ref[pl.ds(row0, m_chunk), :], w,
                preferred_element_type=jnp.float32,
            )
            absmax = jnp.maximum(jnp.max(jnp.abs(part)), 1e-30)
            q_send[k - 1, :, :] = jnp.clip(
                jnp.round(part * (127.0 / absmax)), -127.0, 127.0
            ).astype(jnp.int8)
            s_send[k - 1, :, :] = (absmax * (1.0 / 127.0)) * jnp.ones(
                (1, 128), jnp.float32
            )
            q_rdma = pltpu.make_async_remote_copy(
                src_ref=q_send.at[k - 1],
                dst_ref=q_recv.at[k - 1],
                send_sem=q_send_sems.at[k - 1],
                recv_sem=q_recv_sems.at[k - 1],
                device_id=(dst,),
                device_id_type=pl.DeviceIdType.MESH,
            )
            s_rdma = pltpu.make_async_remote_copy(
                src_ref=s_send.at[k - 1],
                dst_ref=s_recv.at[k - 1],
                send_sem=s_send_sems.at[k - 1],
                recv_sem=s_recv_sems.at[k - 1],
                device_id=(dst,),
                device_id_type=pl.DeviceIdType.MESH,
            )
            q_rdma.start()
            s_rdma.start()
            rdmas.append((q_rdma, s_rdma))

        out_ref[:, :] = jnp.dot(
            x_ref[pl.ds(my * m_chunk, m_chunk), :], w,
            preferred_element_type=jnp.float32,
        )

        for j in range(N_DEV - 1):
            q_rdma, s_rdma = rdmas[j]
            q_rdma.wait_recv()
            s_rdma.wait_recv()
            out_ref[:, :] = out_ref[:, :] + (
                q_recv[j, :, :].astype(jnp.float32) * s_recv[j, 0, 0]
            )

        out_ref[:, :] = jnp.maximum(out_ref[:, :], 0.0)

        for j in range(N_DEV - 1):
            rdmas[j][0].wait_send()
            rdmas[j][1].wait_send()

    return pl.pallas_call(
        body,
        out_shape=jax.ShapeDtypeStruct((m_chunk, n), jnp.float32),
        in_specs=[
            pl.BlockSpec(memory_space=pltpu.VMEM),
            pl.BlockSpec(memory_space=pltpu.VMEM),
        ],
        out_specs=pl.BlockSpec(memory_space=pltpu.VMEM),
        scratch_shapes=[
            pltpu.VMEM((N_DEV - 1, m_chunk, n), jnp.int8),
            pltpu.VMEM((N_DEV - 1, m_chunk, n), jnp.int8),
            pltpu.VMEM((N_DEV - 1, 1, 128), jnp.float32),
            pltpu.VMEM((N_DEV - 1, 1, 128), jnp.float32),
            pltpu.SemaphoreType.DMA((N_DEV - 1,)),
            pltpu.SemaphoreType.DMA((N_DEV - 1,)),
            pltpu.SemaphoreType.DMA((N_DEV - 1,)),
            pltpu.SemaphoreType.DMA((N_DEV - 1,)),
        ],
    )(x, w_mat)
